# baseline (speedup 1.0000x reference)
import functools
import numpy as np
import jax
import jax.numpy as jnp
from jax import lax
from jax.experimental import pallas as pl
from jax.experimental.pallas import tpu as pltpu
from jax.experimental.pallas import tpu_sc as plsc

_W = 4
_R = 8
_NBUF = 4


def _make_sc(Ro, D, rows_used):
    info = plsc.get_sparse_core_info()
    NC, NS = info.num_cores, info.num_subcores
    NW = NC * NS
    rows_per_w = rows_used // NW
    nch = rows_per_w // _R
    laps = nch // _NBUF
    assert nch % _NBUF == 0
    mesh = plsc.VectorSubcoreMesh(core_axis_name="c", subcore_axis_name="s")

    @functools.partial(
        pl.kernel,
        mesh=mesh,
        out_type=jax.ShapeDtypeStruct((Ro, D), jnp.float32),
        scratch_types=(
            [pltpu.VMEM((_R, D), jnp.float32) for _ in range(_NBUF)]
            + [pltpu.VMEM((rows_per_w,), jnp.int32)]
            + [pltpu.SemaphoreType.DMA for _ in range(2 * _NBUF)]
        ),
    )
    def k(x_hbm, idx_hbm, out_hbm, *refs):
        bufs = refs[:_NBUF]
        idx_v = refs[_NBUF]
        sin = refs[_NBUF + 1:_NBUF + 1 + _NBUF]
        sout = refs[_NBUF + 1 + _NBUF:]
        wid = lax.axis_index("s") * NC + lax.axis_index("c")
        base = wid * rows_per_w
        pltpu.sync_copy(idx_hbm.at[pl.ds(base, rows_per_w)], idx_v)

        def start_gather(c, b):
            pltpu.async_copy(
                x_hbm.at[idx_v.at[pl.ds(c * _R, _R)]], bufs[b], sin[b]
            )

        def gather_wait(c, b):
            pltpu.make_async_copy(
                x_hbm.at[idx_v.at[pl.ds(c * _R, _R)]], bufs[b], sin[b]
            ).wait()

        def start_scatter(c, b):
            pltpu.async_copy(bufs[b], out_hbm.at[pl.ds(base + c * _R, _R)], sout[b])

        def scatter_wait(c, b):
            pltpu.make_async_copy(
                bufs[b], out_hbm.at[pl.ds(base + c * _R, _R)], sout[b]
            ).wait()

        for b in range(_NBUF):
            start_gather(b, b)

        def lap(g, _):
            for b in range(_NBUF):
                c = g * _NBUF + b
                gather_wait(c, b)
                start_scatter(c, b)

                @pl.when(g < laps - 1)
                def _():
                    scatter_wait(c, b)
                    start_gather(c + _NBUF, b)

            return 0

        lax.fori_loop(0, laps, lap, 0)
        for b in range(_NBUF):
            scatter_wait(nch - _NBUF + b, b)

    return k


def _tc_body(in_ref, out_ref):
    for k in range(out_ref.shape[0]):
        out_ref[k, :] = in_ref[_W * k, :]


def kernel(x):
    B, S, D = x.shape
    So = S // _W
    nb_sc = B - 1
    rows_sc = nb_sc * So
    x2 = x.reshape(B * S, D)
    idx = np.arange(0, nb_sc * S, _W, dtype=np.int32)
    out_sc = _make_sc(B * So, D, rows_sc)(x2, idx)

    blk = 256
    off = (nb_sc * S) // (blk * _W)
    out_tc = pl.pallas_call(
        _tc_body,
        grid=(S // (blk * _W),),
        in_specs=[pl.BlockSpec((blk * _W, D), lambda i: (i + off, 0))],
        out_specs=pl.BlockSpec((blk, D), lambda i: (i, 0)),
        out_shape=jax.ShapeDtypeStruct((So, D), jnp.float32),
    )(x2)

    out = lax.dynamic_update_slice(out_sc, out_tc, (rows_sc, 0))
    return out.reshape(B, So, D)


# speedup vs baseline: 1.2530x; 1.2530x over previous
import functools
import numpy as np
import jax
import jax.numpy as jnp
from jax import lax
from jax.experimental import pallas as pl
from jax.experimental.pallas import tpu as pltpu
from jax.experimental.pallas import tpu_sc as plsc

_W = 4
_R = 8
_NBUF = 4


def _make_sc(Ro, D):
    info = plsc.get_sparse_core_info()
    NC, NS = info.num_cores, info.num_subcores
    NW = NC * NS
    rows_per_w = Ro // NW
    nch = rows_per_w // _R
    mesh = plsc.VectorSubcoreMesh(core_axis_name="c", subcore_axis_name="s")

    @functools.partial(
        pl.kernel,
        mesh=mesh,
        out_type=jax.ShapeDtypeStruct((Ro, D), jnp.float32),
        scratch_types=(
            [pltpu.VMEM((_R, D), jnp.float32) for _ in range(_NBUF)]
            + [pltpu.VMEM((rows_per_w,), jnp.int32)]
            + [pltpu.SemaphoreType.DMA for _ in range(2 * _NBUF)]
        ),
    )
    def k(x_hbm, idx_hbm, out_hbm, *refs):
        bufs = refs[:_NBUF]
        idx_v = refs[_NBUF]
        sin = refs[_NBUF + 1:_NBUF + 1 + _NBUF]
        sout = refs[_NBUF + 1 + _NBUF:]
        wid = lax.axis_index("s") * NC + lax.axis_index("c")
        base = wid * rows_per_w
        pltpu.sync_copy(idx_hbm.at[pl.ds(base, rows_per_w)], idx_v)

        def start_gather(c, b):
            pltpu.async_copy(
                x_hbm.at[idx_v.at[pl.ds(c * _R, _R)]], bufs[b], sin[b]
            )

        def gather_wait(c, b):
            pltpu.make_async_copy(
                x_hbm.at[idx_v.at[pl.ds(c * _R, _R)]], bufs[b], sin[b]
            ).wait()

        def start_scatter(c, b):
            pltpu.async_copy(bufs[b], out_hbm.at[pl.ds(base + c * _R, _R)], sout[b])

        def scatter_wait(c, b):
            pltpu.make_async_copy(
                bufs[b], out_hbm.at[pl.ds(base + c * _R, _R)], sout[b]
            ).wait()

        for b in range(_NBUF):
            start_gather(b, b)

        def lap(g, _):
            for b in range(_NBUF):
                c = g * _NBUF + b
                gather_wait(c, b)
                start_scatter(c, b)

                @pl.when(g < (nch // _NBUF) - 1)
                def _():
                    scatter_wait(c, b)
                    start_gather(c + _NBUF, b)

            return 0

        lax.fori_loop(0, nch // _NBUF, lap, 0)
        for b in range(_NBUF):
            scatter_wait(nch - _NBUF + b, b)

    return k


def kernel(x):
    B, S, D = x.shape
    So = S // _W
    x2 = x.reshape(B * S, D)
    idx = np.arange(0, B * S, _W, dtype=np.int32)
    out = _make_sc(B * So, D)(x2, idx)
    return out.reshape(B, So, D)
